# baseline (speedup 1.0000x reference)
import functools

import jax
import jax.numpy as jnp
import numpy as np
from jax import lax
from jax.experimental import pallas as pl
from jax.experimental.pallas import tpu as pltpu
from jax.experimental.pallas import tpu_sc as plsc

B, S, E = 4, 4096, 16
TOK = B * S
B_TC = 3
B_SC = B - B_TC
SC_T0 = B_TC * S
SC_TOK = B_SC * S

_INFO = plsc.get_sparse_core_info()
NC, NS, L = _INFO.num_cores, _INFO.num_subcores, _INFO.num_lanes
NW = NC * NS
TPW = SC_TOK // NW
GROUPS = TPW // L

_KS0 = np.uint32(0)
_KS1 = np.uint32(42)
_KS2 = np.uint32(0 ^ 42 ^ 0x1BD11BDA)
_ROT = [[13, 15, 26, 6], [17, 29, 16, 24]]
_KSCHED = [_KS0, _KS1, _KS2]


def _threefry_bits(n):
    x0 = jnp.zeros(n.shape, dtype=jnp.uint32) + _KS0
    x1 = n + _KS1
    for i in range(5):
        for r in _ROT[i % 2]:
            x0 = x0 + x1
            x1 = (x1 << np.uint32(r)) | (x1 >> np.uint32(32 - r))
            x1 = x0 ^ x1
        x0 = x0 + _KSCHED[(i + 1) % 3]
        x1 = x1 + _KSCHED[(i + 2) % 3] + np.uint32(i + 1)
    return x0 ^ x1


def _tc_body(oh_ref, ones_ref):
    b_i = jax.lax.broadcasted_iota(jnp.uint32, (B_TC, E, S), 0)
    e_i = jax.lax.broadcasted_iota(jnp.uint32, (B_TC, E, S), 1)
    s_i = jax.lax.broadcasted_iota(jnp.uint32, (B_TC, E, S), 2)
    n = b_i * np.uint32(S * E) + s_i * np.uint32(E) + e_i
    bits = (_threefry_bits(n) >> np.uint32(9)).astype(jnp.int32)
    mx = jnp.max(bits, axis=1, keepdims=True)
    oh = (bits == mx).astype(jnp.float32)
    oh_ref[...] = jnp.swapaxes(oh, 1, 2)
    ones_ref[...] = jnp.ones((TOK,), dtype=jnp.float32)


_tc_router = functools.partial(
    pl.pallas_call,
    out_shape=(
        jax.ShapeDtypeStruct((B_TC, S, E), jnp.float32),
        jax.ShapeDtypeStruct((TOK,), jnp.float32),
    ),
)(_tc_body)


def _sc_body(oh_hbm, oh_v, dma_sem):
    wid = lax.axis_index("s") * NC + lax.axis_index("c")
    base = SC_T0 + wid * TPW
    lane = lax.iota(jnp.int32, L)
    zeros16 = jnp.zeros((L,), dtype=jnp.float32)
    ones16 = jnp.ones((L,), dtype=jnp.float32)

    def group(g, carry):
        tok = (base + g * L + lane).astype(jnp.uint32)
        best = None
        best_e = None
        for e in range(E):
            n = tok * np.uint32(E) + np.uint32(e)
            bits = (_threefry_bits(n) >> np.uint32(9)).astype(jnp.int32)
            if e == 0:
                best = bits
                best_e = jnp.zeros((L,), dtype=jnp.int32)
            else:
                gt = bits > best
                best = jnp.where(gt, bits, best)
                best_e = jnp.where(gt, jnp.full((L,), e, dtype=jnp.int32), best_e)
        row0 = g * L
        for r in range(0, L * E, L):
            oh_v[pl.ds(row0 * E + r, L)] = zeros16
        plsc.store_scatter(oh_v, [(row0 + lane) * E + best_e], ones16)
        return carry

    lax.fori_loop(0, GROUPS, group, 0)

    pltpu.async_copy(oh_v, oh_hbm.at[pl.ds(wid * TPW * E, TPW * E)], dma_sem).wait()


_sc_router = functools.partial(
    pl.kernel,
    out_type=jax.ShapeDtypeStruct((SC_TOK * E,), jnp.float32),
    mesh=plsc.VectorSubcoreMesh(core_axis_name="c", subcore_axis_name="s"),
    compiler_params=pltpu.CompilerParams(needs_layout_passes=False),
    scratch_types=[
        pltpu.VMEM((TPW * E,), jnp.float32),
        pltpu.SemaphoreType.DMA,
    ],
)(_sc_body)


def kernel(x):
    del x
    tc_oh, ones = _tc_router()
    sc_oh = _sc_router()
    one_hot = jnp.concatenate([tc_oh, sc_oh.reshape(B_SC, S, E)], axis=0)
    return (one_hot, ones.reshape(B, S, 1), one_hot)
